# Initial kernel scaffold; baseline (speedup 1.0000x reference)
import jax
import jax.numpy as jnp
from jax.experimental import pallas as pl


def kernel(v1, v2, W_v1_to_v2, W_v1_to_v4, W_v2_to_v4):
    raise NotImplementedError("write your pallas kernel here")


# speedup vs baseline: 4.0718x; 4.0718x over previous
import functools

import jax
import jax.numpy as jnp
from jax import lax
from jax.experimental import pallas as pl
from jax.experimental.pallas import tpu as pltpu
from jax.experimental.pallas import tpu_sc as plsc

_B = 64
_NC = 2
_NS = 16
_NW = _NC * _NS


def _make_axon():
    mesh = plsc.VectorSubcoreMesh(core_axis_name="c", subcore_axis_name="s")

    @functools.partial(
        pl.kernel,
        mesh=mesh,
        compiler_params=pltpu.CompilerParams(needs_layout_passes=False),
        out_type=[
            jax.ShapeDtypeStruct((_B, 256, 256), jnp.float32),
            jax.ShapeDtypeStruct((_B, 128, 128), jnp.float32),
        ],
        scratch_types=[
            pltpu.VMEM((8, 256), jnp.float32),
            pltpu.VMEM((4, 128), jnp.float32),
            pltpu.VMEM((4, 128), jnp.float32),
            pltpu.VMEM((4096,), jnp.float32),
            pltpu.VMEM((1024,), jnp.float32),
            pltpu.VMEM((8, 256), jnp.float32),
            pltpu.VMEM((4, 128), jnp.float32),
            pltpu.SemaphoreType.DMA,
            pltpu.SemaphoreType.DMA,
        ],
    )
    def axon(v1_hbm, v2_hbm, w12_hbm, w14_hbm, w24_hbm, o2_hbm, o4_hbm,
             w12_v, w14_v, w24_v, v1r_v, v2r_v, o2r_v, o4r_v, lsem, ssem):
        w = lax.axis_index("s") * _NC + lax.axis_index("c")
        pltpu.sync_copy(w12_hbm.at[pl.ds(8 * w, 8)], w12_v)
        pltpu.sync_copy(w14_hbm.at[pl.ds(4 * w, 4)], w14_v)
        pltpu.sync_copy(w24_hbm.at[pl.ds(4 * w, 4)], w24_v)
        iota = lax.iota(jnp.int32, 16)
        idx2 = iota * 2
        idx4 = iota * 4

        def body(b, carry):
            loads = []
            for r in range(8):
                loads.append(pltpu.async_copy(
                    v1_hbm.at[b, 16 * w + 2 * r],
                    v1r_v.at[pl.ds(512 * r, 512)], lsem))
            for t in range(4):
                loads.append(pltpu.async_copy(
                    v2_hbm.at[b, 8 * w + 2 * t],
                    v2r_v.at[pl.ds(256 * t, 256)], lsem))
            for h in loads:
                h.wait()
            for r in range(8):
                for k in range(16):
                    x = plsc.load_gather(v1r_v, [idx2 + (512 * r + 32 * k)])
                    o2r_v[r, pl.ds(16 * k, 16)] = x * w12_v[r, pl.ds(16 * k, 16)]
            for t in range(4):
                for k in range(8):
                    xa = plsc.load_gather(v1r_v, [idx4 + (1024 * t + 64 * k)])
                    xb = plsc.load_gather(v2r_v, [idx2 + (256 * t + 32 * k)])
                    o4r_v[t, pl.ds(16 * k, 16)] = (
                        xa * w14_v[t, pl.ds(16 * k, 16)]
                        + xb * w24_v[t, pl.ds(16 * k, 16)])
            s1 = pltpu.async_copy(o2r_v, o2_hbm.at[b, pl.ds(8 * w, 8)], ssem)
            s2 = pltpu.async_copy(o4r_v, o4_hbm.at[b, pl.ds(4 * w, 4)], ssem)
            s1.wait()
            s2.wait()
            return carry

        lax.fori_loop(0, _B, body, 0)

    return axon


_axon = _make_axon()


def kernel(v1, v2, W_v1_to_v2, W_v1_to_v4, W_v2_to_v4):
    return tuple(_axon(v1, v2, W_v1_to_v2, W_v1_to_v4, W_v2_to_v4))


# speedup vs baseline: 5.3210x; 1.3068x over previous
import functools

import jax
import jax.numpy as jnp
from jax import lax
from jax.experimental import pallas as pl
from jax.experimental.pallas import tpu as pltpu
from jax.experimental.pallas import tpu_sc as plsc

_B = 64
_NC = 2
_NS = 16


def _make_axon():
    mesh = plsc.VectorSubcoreMesh(core_axis_name="c", subcore_axis_name="s")

    @functools.partial(
        pl.kernel,
        mesh=mesh,
        compiler_params=pltpu.CompilerParams(needs_layout_passes=False),
        out_type=[
            jax.ShapeDtypeStruct((_B, 256, 256), jnp.float32),
            jax.ShapeDtypeStruct((_B, 128, 128), jnp.float32),
        ],
        scratch_types=[
            pltpu.VMEM((8, 256), jnp.float32),
            pltpu.VMEM((4, 128), jnp.float32),
            pltpu.VMEM((4, 128), jnp.float32),
            pltpu.VMEM((2, 4096), jnp.float32),
            pltpu.VMEM((2, 1024), jnp.float32),
            pltpu.VMEM((2, 8, 256), jnp.float32),
            pltpu.VMEM((2, 4, 128), jnp.float32),
            pltpu.SemaphoreType.DMA,
            pltpu.SemaphoreType.DMA,
            pltpu.SemaphoreType.DMA,
        ],
    )
    def axon(v1_hbm, v2_hbm, w12_hbm, w14_hbm, w24_hbm, o2_hbm, o4_hbm,
             w12_v, w14_v, w24_v, v1r_v, v2r_v, o2r_v, o4r_v,
             lsem, ssem0, ssem1):
        w = lax.axis_index("s") * _NC + lax.axis_index("c")
        pltpu.sync_copy(w12_hbm.at[pl.ds(8 * w, 8)], w12_v)
        pltpu.sync_copy(w14_hbm.at[pl.ds(4 * w, 4)], w14_v)
        pltpu.sync_copy(w24_hbm.at[pl.ds(4 * w, 4)], w24_v)
        iota = lax.iota(jnp.int32, 16)
        idx2 = iota * 2
        idx4 = iota * 4

        def load_descs(b, buf):
            descs = []
            for r in range(8):
                descs.append((v1_hbm.at[b, 16 * w + 2 * r],
                              v1r_v.at[buf, pl.ds(512 * r, 512)]))
            for t in range(4):
                descs.append((v2_hbm.at[b, 8 * w + 2 * t],
                              v2r_v.at[buf, pl.ds(256 * t, 256)]))
            return descs

        def issue_loads(b, buf):
            for src, dst in load_descs(b, buf):
                pltpu.async_copy(src, dst, lsem)

        def wait_loads(buf):
            for src, dst in load_descs(0, buf):
                pltpu.make_async_copy(src, dst, lsem).wait()

        def store_descs(b, buf):
            return (
                (o2r_v.at[buf], o2_hbm.at[b, pl.ds(8 * w, 8)]),
                (o4r_v.at[buf], o4_hbm.at[b, pl.ds(4 * w, 4)]),
            )

        def issue_stores(b, buf, sem):
            for src, dst in store_descs(b, buf):
                pltpu.async_copy(src, dst, sem)

        def wait_stores(buf, sem):
            for src, dst in store_descs(0, buf):
                pltpu.make_async_copy(src, dst, sem).wait()

        def compute(buf):
            for r in range(8):
                for k in range(16):
                    sl = pl.ds(16 * k, 16)
                    x = plsc.load_gather(
                        v1r_v, [jnp.full((16,), buf, jnp.int32),
                                idx2 + (512 * r + 32 * k)])
                    o2r_v[buf, r, sl] = x * w12_v[r, sl]
            for t in range(4):
                for k in range(8):
                    sl = pl.ds(16 * k, 16)
                    xa = plsc.load_gather(
                        v1r_v, [jnp.full((16,), buf, jnp.int32),
                                idx4 + (1024 * t + 64 * k)])
                    xb = plsc.load_gather(
                        v2r_v, [jnp.full((16,), buf, jnp.int32),
                                idx2 + (256 * t + 32 * k)])
                    o4r_v[buf, t, sl] = (xa * w14_v[t, sl]
                                         + xb * w24_v[t, sl])

        issue_loads(0, 0)

        def body(i, carry):
            b0 = 2 * i
            b1 = b0 + 1
            issue_loads(b1, 1)
            wait_loads(0)

            @pl.when(i >= 1)
            def _():
                wait_stores(0, ssem0)

            compute(0)
            issue_stores(b0, 0, ssem0)
            issue_loads(jnp.minimum(b0 + 2, _B - 1), 0)
            wait_loads(1)

            @pl.when(i >= 1)
            def _():
                wait_stores(1, ssem1)

            compute(1)
            issue_stores(b1, 1, ssem1)
            return carry

        lax.fori_loop(0, _B // 2, body, 0)
        wait_loads(0)
        wait_stores(0, ssem0)
        wait_stores(1, ssem1)

    return axon


_axon = _make_axon()


def kernel(v1, v2, W_v1_to_v2, W_v1_to_v4, W_v2_to_v4):
    return tuple(_axon(v1, v2, W_v1_to_v2, W_v1_to_v4, W_v2_to_v4))
